# baseline (speedup 1.0000x reference)
import functools

import jax
import jax.numpy as jnp
from jax import lax
from jax.experimental import pallas as pl
from jax.experimental.pallas import tpu as pltpu
from jax.experimental.pallas import tpu_sc as plsc

VOCAB = 100000
MAXLEN = 200
D = 64
BATCH = 4096

NC = 2
NS = 16
L = 16
NW = NC * NS
SPW = BATCH // NW
HALF = MAXLEN // 2
NBUF = 3
GRP = D // L
ROW_UNROLL = 4

_mesh = plsc.VectorSubcoreMesh(
    core_axis_name="c", subcore_axis_name="s", num_cores=NC, num_subcores=NS
)


@functools.partial(
    pl.kernel,
    mesh=_mesh,
    out_type=jax.ShapeDtypeStruct((BATCH, MAXLEN, D), jnp.float32),
    scratch_types=[
        pltpu.VMEM((SPW, 2, HALF), jnp.int32),
        pltpu.VMEM((MAXLEN, D), jnp.float32),
        pltpu.VMEM((NBUF, MAXLEN, D), jnp.float32),
        pltpu.VMEM((NBUF, MAXLEN, D), jnp.float32),
        pltpu.SemaphoreType.DMA((NBUF,)),
        pltpu.SemaphoreType.DMA((NBUF,)),
    ],
    compiler_params=pltpu.CompilerParams(use_tc_tiling_on_sc=False),
)
def _emb(xr, pos, tok, out, idx_v, pos_v, gbuf, sbuf, gsem, ssem):
    wid = lax.axis_index("s") * NC + lax.axis_index("c")
    s0 = wid * SPW

    pltpu.sync_copy(xr.at[pl.ds(s0, SPW)], idx_v)
    pltpu.sync_copy(pos, pos_v)

    def g_descs(c, b):
        return [
            pltpu.make_async_copy(
                tok.at[idx_v.at[c, h]],
                gbuf.at[b, pl.ds(h * HALF, HALF)],
                gsem.at[b],
            )
            for h in range(2)
        ]

    def s_desc(c, b):
        return pltpu.make_async_copy(sbuf.at[b], out.at[s0 + c], ssem.at[b])

    def process(c, b, first, n_regather):
        for d in g_descs(c, b):
            d.wait()
        if not first:
            s_desc(c - NBUF, b).wait()
        gb = gbuf.at[b]
        sb = sbuf.at[b]

        def row_body(r, carry):
            for u in range(ROW_UNROLL):
                rr = r * ROW_UNROLL + u
                for g in range(GRP):
                    sl = pl.ds(g * L, L)
                    sb[rr, sl] = gb[rr, sl] + pos_v[rr, sl]
            return carry

        lax.fori_loop(0, MAXLEN // ROW_UNROLL, row_body, 0)
        s_desc(c, b).start()
        if n_regather:
            for d in g_descs(c + NBUF, b):
                d.start()

    for b in range(NBUF):
        for d in g_descs(b, b):
            d.start()
    for b in range(NBUF):
        process(b, b, first=True, n_regather=True)

    MAIN = (SPW - 2 * NBUF) // NBUF * NBUF

    def main(i, carry):
        for u in range(NBUF):
            c = NBUF + i * NBUF + u
            process(c, u, first=False, n_regather=True)
        return carry

    lax.fori_loop(0, MAIN // NBUF, main, 0)

    for c in range(NBUF + MAIN, SPW):
        process(c, c % NBUF, first=False, n_regather=(c + NBUF < SPW))
    for c in range(SPW - NBUF, SPW):
        s_desc(c, c % NBUF).wait()


def kernel(x, token_table, pos_table):
    xr = x.astype(jnp.int32).reshape(BATCH, 2, HALF)
    return _emb(xr, pos_table, token_table)
